# baseline (speedup 1.0000x reference)
import jax
import jax.numpy as jnp
from jax import lax
from jax.experimental import pallas as pl
from jax.experimental.pallas import tpu as pltpu
from jax.experimental.pallas import tpu_sc as plsc

B = 4
B_SC = 1
S = 2048
D = 1024
NC = 2
NS = 16
NW = NC * NS
SPW = S // NW
CH = 16
NCHUNK = SPW // CH
NSTAGE = NCHUNK * B_SC
NVEC = D // 16
BS_TC = 256


def _sc_body(x_hbm, emb_hbm, out_hbm,
             eb0, eb1, xb0, xb1,
             esem0, esem1, lsem0, lsem1, ssem0, ssem1):
    wid = lax.axis_index("s") * NC + lax.axis_index("c")
    s0 = wid * SPW
    ebufs, esems = (eb0, eb1), (esem0, esem1)
    xbufs, lsems, ssems = (xb0, xb1), (lsem0, lsem1), (ssem0, ssem1)

    def soff(c):
        return s0 + c * CH

    eload = {0: pltpu.async_copy(emb_hbm.at[pl.ds(soff(0), CH)], eb0, esem0)}
    xload = {0: pltpu.async_copy(x_hbm.at[0, pl.ds(soff(0), CH)], xb0, lsem0)}
    store = {}

    for t in range(NSTAGE):
        c, b = divmod(t, B_SC)
        pb = t % 2
        if b == 0 and c + 1 < NCHUNK:
            ne = (c + 1) % 2
            eload[c + 1] = pltpu.async_copy(
                emb_hbm.at[pl.ds(soff(c + 1), CH)], ebufs[ne], esems[ne])
        if t + 1 < NSTAGE:
            if t - 1 in store:
                store.pop(t - 1).wait()
            nc_, nb_ = divmod(t + 1, B_SC)
            np_ = (t + 1) % 2
            xload[t + 1] = pltpu.async_copy(
                x_hbm.at[nb_, pl.ds(soff(nc_), CH)], xbufs[np_], lsems[np_])
        xload.pop(t).wait()
        if b == 0:
            eload.pop(c).wait()

        xbuf, ebuf = xbufs[pb], ebufs[c % 2]

        def row_body(r, rc, xbuf=xbuf, ebuf=ebuf):
            for j in range(NVEC):
                sl = pl.ds(j * 16, 16)
                xbuf[r, sl] = xbuf[r, sl] + ebuf[r, sl]
            return rc

        lax.fori_loop(0, CH, row_body, 0)
        store[t] = pltpu.async_copy(
            xbuf, out_hbm.at[b, pl.ds(soff(c), CH)], ssems[pb])

    for h in store.values():
        h.wait()


def _sc_call(x, emb_weight):
    mesh = plsc.VectorSubcoreMesh(core_axis_name="c", subcore_axis_name="s")
    f = pl.kernel(
        _sc_body,
        out_type=jax.ShapeDtypeStruct((B_SC, S, D), jnp.float32),
        mesh=mesh,
        scratch_types=[
            pltpu.VMEM((CH, D), jnp.float32),
            pltpu.VMEM((CH, D), jnp.float32),
            pltpu.VMEM((CH, D), jnp.float32),
            pltpu.VMEM((CH, D), jnp.float32),
            pltpu.SemaphoreType.DMA,
            pltpu.SemaphoreType.DMA,
            pltpu.SemaphoreType.DMA,
            pltpu.SemaphoreType.DMA,
            pltpu.SemaphoreType.DMA,
            pltpu.SemaphoreType.DMA,
        ],
    )
    return f(x, emb_weight)


def _tc_add_body(x_ref, e_ref, o_ref):
    o_ref[...] = x_ref[...] + e_ref[...]


def _tc_call(x, emb_weight):
    grid = (S // BS_TC, B - B_SC)
    return pl.pallas_call(
        _tc_add_body,
        grid=grid,
        in_specs=[
            pl.BlockSpec((1, BS_TC, D), lambda s, b: (b + B_SC, s, 0)),
            pl.BlockSpec((BS_TC, D), lambda s, b: (s, 0)),
        ],
        out_specs=pl.BlockSpec((1, BS_TC, D), lambda s, b: (b, s, 0)),
        out_shape=jax.ShapeDtypeStruct((B - B_SC, S, D), x.dtype),
    )(x, emb_weight)


def kernel(x, emb_weight):
    sc_out = _sc_call(x, emb_weight)
    tc_out = _tc_call(x, emb_weight)
    return jnp.concatenate([sc_out, tc_out], axis=0)


# speedup vs baseline: 1.2342x; 1.2342x over previous
import jax
import jax.numpy as jnp
from jax import lax
from jax.experimental import pallas as pl
from jax.experimental.pallas import tpu as pltpu
from jax.experimental.pallas import tpu_sc as plsc

B = 4
S = 2048
D = 1024
NC = 2
NS = 16
NW = NC * NS
SPW = S // NW
CH = 16
NCHUNK = SPW // CH
NSTAGE = NCHUNK * B
NVEC = D // 16
NXB = 4


def _sc_body(x_hbm, emb_hbm, out_hbm,
             eb0, eb1, xb0, xb1, xb2, xb3,
             esem0, esem1, ls0, ls1, ls2, ls3, ss0, ss1, ss2, ss3):
    wid = lax.axis_index("s") * NC + lax.axis_index("c")
    s0 = wid * SPW
    ebufs, esems = (eb0, eb1), (esem0, esem1)
    xbufs = (xb0, xb1, xb2, xb3)
    lsems = (ls0, ls1, ls2, ls3)
    ssems = (ss0, ss1, ss2, ss3)

    def soff(c):
        return s0 + c * CH

    def start_load(t):
        c, b = divmod(t, B)
        return pltpu.async_copy(
            x_hbm.at[b, pl.ds(soff(c), CH)], xbufs[t % NXB], lsems[t % NXB])

    eload = {0: pltpu.async_copy(emb_hbm.at[pl.ds(soff(0), CH)], eb0, esem0)}
    xload = {t: start_load(t) for t in range(2)}
    store = {}

    for t in range(NSTAGE):
        c, b = divmod(t, B)
        if b == 0 and c + 1 < NCHUNK:
            ne = (c + 1) % 2
            eload[c + 1] = pltpu.async_copy(
                emb_hbm.at[pl.ds(soff(c + 1), CH)], ebufs[ne], esems[ne])
        if t + 2 < NSTAGE:
            if t - 2 in store:
                store.pop(t - 2).wait()
            xload[t + 2] = start_load(t + 2)
        xload.pop(t).wait()
        if b == 0:
            eload.pop(c).wait()

        xbuf, ebuf = xbufs[t % NXB], ebufs[c % 2]

        def row_body(r, rc, xbuf=xbuf, ebuf=ebuf):
            for j in range(NVEC):
                sl = pl.ds(j * 16, 16)
                xbuf[r, sl] = xbuf[r, sl] + ebuf[r, sl]
            return rc

        lax.fori_loop(0, CH, row_body, 0)
        store[t] = pltpu.async_copy(
            xbuf, out_hbm.at[b, pl.ds(soff(c), CH)], ssems[t % NXB])

    for h in store.values():
        h.wait()


def kernel(x, emb_weight):
    mesh = plsc.VectorSubcoreMesh(core_axis_name="c", subcore_axis_name="s")
    f = pl.kernel(
        _sc_body,
        out_type=jax.ShapeDtypeStruct((B, S, D), jnp.float32),
        mesh=mesh,
        scratch_types=[
            pltpu.VMEM((CH, D), jnp.float32),
            pltpu.VMEM((CH, D), jnp.float32),
            pltpu.VMEM((CH, D), jnp.float32),
            pltpu.VMEM((CH, D), jnp.float32),
            pltpu.VMEM((CH, D), jnp.float32),
            pltpu.VMEM((CH, D), jnp.float32),
            pltpu.SemaphoreType.DMA,
            pltpu.SemaphoreType.DMA,
            pltpu.SemaphoreType.DMA,
            pltpu.SemaphoreType.DMA,
            pltpu.SemaphoreType.DMA,
            pltpu.SemaphoreType.DMA,
            pltpu.SemaphoreType.DMA,
            pltpu.SemaphoreType.DMA,
            pltpu.SemaphoreType.DMA,
            pltpu.SemaphoreType.DMA,
        ],
    )
    return f(x, emb_weight)
